# baseline (speedup 1.0000x reference)
import functools
import jax
import jax.numpy as jnp
from jax import lax
from jax.experimental import pallas as pl
from jax.experimental.pallas import tpu as pltpu
from jax.experimental.pallas import tpu_sc as plsc

_LANES = 16


def _make_sc_kernel(bs, h, w, nf):
    C = 2 * nf
    NC, NS = 2, 16
    NW = NC * NS
    assert h == NW, "one y row per vector subcore"

    mesh = plsc.VectorSubcoreMesh(core_axis_name="c", subcore_axis_name="s")

    @functools.partial(
        pl.kernel,
        out_type=jax.ShapeDtypeStruct((bs, h, w, C), jnp.float32),
        mesh=mesh,
        scratch_types=[
            pltpu.VMEM((w, nf), jnp.float32),
            pltpu.VMEM((1, nf), jnp.float32),
            pltpu.VMEM((w, C), jnp.float32),
            pltpu.SemaphoreType.DMA,
        ],
        compiler_params=pltpu.CompilerParams(needs_layout_passes=False),
    )
    def sc_kernel(col_hbm, row_hbm, out_hbm, col_v, row_v, block_v, sem):
        y = lax.axis_index("s") * NC + lax.axis_index("c")
        pltpu.sync_copy(col_hbm.at[pl.ds(0, w)], col_v)
        pltpu.sync_copy(row_hbm.at[pl.ds(y, 1)], row_v)

        def build(x, carry):
            for j in range(nf // _LANES):
                block_v[x, pl.ds(j * _LANES, _LANES)] = col_v[
                    x, pl.ds(j * _LANES, _LANES)
                ]
                block_v[x, pl.ds(nf + j * _LANES, _LANES)] = row_v[
                    0, pl.ds(j * _LANES, _LANES)
                ]
            return carry

        half = w // 2
        lax.fori_loop(0, half, build, 0)
        copies = [
            pltpu.async_copy(
                block_v.at[pl.ds(0, half)],
                out_hbm.at[b, y, pl.ds(0, half)],
                sem,
            )
            for b in range(bs)
        ]
        lax.fori_loop(half, w, build, 0)
        copies += [
            pltpu.async_copy(
                block_v.at[pl.ds(half, w - half)],
                out_hbm.at[b, y, pl.ds(half, w - half)],
                sem,
            )
            for b in range(bs)
        ]
        for cp in copies:
            cp.wait()

    return sc_kernel


def kernel(mask, row_embed, col_embed):
    bs = mask.shape[0]
    h, w = mask.shape[-2:]
    nf = row_embed.shape[1]
    out = _make_sc_kernel(bs, h, w, nf)(col_embed, row_embed)
    return out.transpose(0, 3, 1, 2)


# speedup vs baseline: 1.0028x; 1.0028x over previous
import functools
import jax
import jax.numpy as jnp
from jax import lax
from jax.experimental import pallas as pl
from jax.experimental.pallas import tpu as pltpu
from jax.experimental.pallas import tpu_sc as plsc

_LANES = 16


def _make_sc_kernel(bs, h, w, nf):
    C = 2 * nf
    NC, NS = 2, 16
    NW = NC * NS
    assert h == NW, "one y row per vector subcore"

    mesh = plsc.VectorSubcoreMesh(core_axis_name="c", subcore_axis_name="s")

    @functools.partial(
        pl.kernel,
        out_type=jax.ShapeDtypeStruct((bs, h, w, C), jnp.float32),
        mesh=mesh,
        scratch_types=[
            pltpu.VMEM((w, nf), jnp.float32),
            pltpu.VMEM((1, nf), jnp.float32),
            pltpu.VMEM((w, C), jnp.float32),
            pltpu.SemaphoreType.DMA,
        ],
    )
    def sc_kernel(col_hbm, row_hbm, out_hbm, col_v, row_v, block_v, sem):
        y = lax.axis_index("s") * NC + lax.axis_index("c")
        pltpu.sync_copy(col_hbm.at[pl.ds(0, w)], col_v)
        pltpu.sync_copy(row_hbm.at[pl.ds(y, 1)], row_v)

        def build(x, carry):
            for j in range(nf // _LANES):
                block_v[x, pl.ds(j * _LANES, _LANES)] = col_v[
                    x, pl.ds(j * _LANES, _LANES)
                ]
                block_v[x, pl.ds(nf + j * _LANES, _LANES)] = row_v[
                    0, pl.ds(j * _LANES, _LANES)
                ]
            return carry

        half = w // 2
        lax.fori_loop(0, half, build, 0)
        copies = [
            pltpu.async_copy(
                block_v.at[pl.ds(0, half)],
                out_hbm.at[b, y, pl.ds(0, half)],
                sem,
            )
            for b in range(bs)
        ]
        lax.fori_loop(half, w, build, 0)
        copies += [
            pltpu.async_copy(
                block_v.at[pl.ds(half, w - half)],
                out_hbm.at[b, y, pl.ds(half, w - half)],
                sem,
            )
            for b in range(bs)
        ]
        for cp in copies:
            cp.wait()

    return sc_kernel


def kernel(mask, row_embed, col_embed):
    bs = mask.shape[0]
    h, w = mask.shape[-2:]
    nf = row_embed.shape[1]
    out = _make_sc_kernel(bs, h, w, nf)(col_embed, row_embed)
    return out.transpose(0, 3, 1, 2)


# speedup vs baseline: 1.0158x; 1.0129x over previous
import functools
import jax
import jax.numpy as jnp
from jax import lax
from jax.experimental import pallas as pl
from jax.experimental.pallas import tpu as pltpu
from jax.experimental.pallas import tpu_sc as plsc

_LANES = 16


def _make_sc_kernel(bs, h, w, nf):
    C = 2 * nf
    NC, NS = 2, 16
    NW = NC * NS
    assert h == NW, "one y row per vector subcore"

    mesh = plsc.VectorSubcoreMesh(core_axis_name="c", subcore_axis_name="s")

    @functools.partial(
        pl.kernel,
        out_type=jax.ShapeDtypeStruct((bs, h, w, C), jnp.float32),
        mesh=mesh,
        scratch_types=[
            pltpu.VMEM((w, nf), jnp.float32),
            pltpu.VMEM((1, nf), jnp.float32),
            pltpu.VMEM((w, C), jnp.float32),
            pltpu.SemaphoreType.DMA,
        ],
        compiler_params=pltpu.CompilerParams(skip_device_barrier=True),
    )
    def sc_kernel(col_hbm, row_hbm, out_hbm, col_v, row_v, block_v, sem):
        y = lax.axis_index("s") * NC + lax.axis_index("c")
        pltpu.sync_copy(col_hbm.at[pl.ds(0, w)], col_v)
        pltpu.sync_copy(row_hbm.at[pl.ds(y, 1)], row_v)

        def build(x, carry):
            for j in range(nf // _LANES):
                block_v[x, pl.ds(j * _LANES, _LANES)] = col_v[
                    x, pl.ds(j * _LANES, _LANES)
                ]
                block_v[x, pl.ds(nf + j * _LANES, _LANES)] = row_v[
                    0, pl.ds(j * _LANES, _LANES)
                ]
            return carry

        half = w // 2
        lax.fori_loop(0, half, build, 0)
        copies = [
            pltpu.async_copy(
                block_v.at[pl.ds(0, half)],
                out_hbm.at[b, y, pl.ds(0, half)],
                sem,
            )
            for b in range(bs)
        ]
        lax.fori_loop(half, w, build, 0)
        copies += [
            pltpu.async_copy(
                block_v.at[pl.ds(half, w - half)],
                out_hbm.at[b, y, pl.ds(half, w - half)],
                sem,
            )
            for b in range(bs)
        ]
        for cp in copies:
            cp.wait()

    return sc_kernel


def kernel(mask, row_embed, col_embed):
    bs = mask.shape[0]
    h, w = mask.shape[-2:]
    nf = row_embed.shape[1]
    out = _make_sc_kernel(bs, h, w, nf)(col_embed, row_embed)
    return out.transpose(0, 3, 1, 2)
